# baseline (speedup 1.0000x reference)
import functools

import jax
import jax.numpy as jnp
from jax import lax
from jax.experimental import pallas as pl
from jax.experimental.pallas import tpu as pltpu
from jax.experimental.pallas import tpu_sc as plsc

_NC = 2
_NS = 16
_NW = _NC * _NS
_NSLOT = 3


@functools.lru_cache(maxsize=None)
def _sc_edge_diff(E, N, D):
    epw = E // _NW
    C = 80
    n_chunks = epw // C
    stripe = (N // _NS) // 8 * 8
    last_stripe = N - (_NS - 1) * stripe

    mesh = plsc.VectorSubcoreMesh(core_axis_name="c", subcore_axis_name="s")

    @functools.partial(
        pl.kernel,
        mesh=mesh,
        out_type=jax.ShapeDtypeStruct((E, D), jnp.float32),
        scratch_types=[
            pltpu.VMEM((epw,), jnp.int32),
            pltpu.VMEM((epw,), jnp.int32),
            pltpu.VMEM((_NSLOT, C, D), jnp.float32),
            pltpu.VMEM_SHARED((N, D), jnp.float32),
            pltpu.SemaphoreType.DMA,
            pltpu.SemaphoreType.DMA,
        ] + [pltpu.SemaphoreType.DMA] * (3 * _NSLOT),
    )
    def k(x_hbm, xneg_hbm, src_hbm, dst_hbm, out_hbm,
          idx_s, idx_d, buf, xneg_sp, sem_is, sem_id, *sems):
        sem_g1 = sems[0:_NSLOT]
        sem_g2 = sems[_NSLOT:2 * _NSLOT]
        sem_o = sems[2 * _NSLOT:3 * _NSLOT]

        wid = lax.axis_index("s") * _NC + lax.axis_index("c")
        base = wid * epw

        cp_is = pltpu.async_copy(src_hbm.at[pl.ds(base, epw)], idx_s, sem_is)
        cp_id = pltpu.async_copy(dst_hbm.at[pl.ds(base, epw)], idx_d, sem_id)

        sid = lax.axis_index("s")

        @pl.when(sid < _NS - 1)
        def _():
            pltpu.sync_copy(xneg_hbm.at[pl.ds(sid * stripe, stripe)],
                            xneg_sp.at[pl.ds(sid * stripe, stripe)])

        @pl.when(sid == _NS - 1)
        def _():
            pltpu.sync_copy(xneg_hbm.at[pl.ds((_NS - 1) * stripe, last_stripe)],
                            xneg_sp.at[pl.ds((_NS - 1) * stripe, last_stripe)])

        plsc.subcore_barrier()

        cp_is.wait()
        cp_id.wait()

        def issue_g1(c, b):
            pltpu.async_copy(
                xneg_sp.at[idx_s.at[pl.ds(c * C, C)]], buf.at[b], sem_g1[b])

        def wait_g1(b):
            pltpu.make_async_copy(
                xneg_sp.at[idx_s.at[pl.ds(0, C)]], buf.at[b], sem_g1[b]).wait()

        def issue_g2(c, b):
            pltpu.async_copy(
                x_hbm.at[idx_d.at[pl.ds(c * C, C)]], buf.at[b], sem_g2[b],
                add=True)

        def wait_g2(b):
            pltpu.make_async_copy(
                x_hbm.at[idx_d.at[pl.ds(0, C)]], buf.at[b], sem_g2[b]).wait()

        def issue_write(c, b):
            pltpu.async_copy(
                buf.at[b], out_hbm.at[pl.ds(base + c * C, C)], sem_o[b])

        def wait_write(b):
            pltpu.make_async_copy(
                buf.at[b], out_hbm.at[pl.ds(base, C)], sem_o[b]).wait()

        issue_g1(0, 0)
        issue_g1(1, 1)
        wait_g1(0)
        issue_g2(0, 0)

        def stage(c, b):
            wait_g2(b)
            issue_write(c, b)

            b2 = (b + 2) % _NSLOT

            @pl.when(c + 2 < n_chunks)
            def _():
                @pl.when(c + 2 >= _NSLOT)
                def _():
                    wait_write(b2)
                issue_g1(c + 2, b2)

            b1 = (b + 1) % _NSLOT

            @pl.when(c + 1 < n_chunks)
            def _():
                wait_g1(b1)
                issue_g2(c + 1, b1)

        n_pipe = n_chunks - (n_chunks % _NSLOT)

        def body(i, carry):
            for b in range(_NSLOT):
                stage(i * _NSLOT + b, b)
            return carry

        lax.fori_loop(0, n_pipe // _NSLOT, body, 0)
        for c in range(n_pipe, n_chunks):
            stage(c, c % _NSLOT)

        for b in range(min(_NSLOT, n_chunks)):
            wait_write(b)

    return k


def kernel(x, edge_index):
    N, D = x.shape
    E = edge_index.shape[1]
    src = edge_index[0]
    dst = edge_index[1]
    return _sc_edge_diff(E, N, D)(x, -x, src, dst)


# speedup vs baseline: 1.1427x; 1.1427x over previous
import functools

import jax
import jax.numpy as jnp
from jax import lax
from jax.experimental import pallas as pl
from jax.experimental.pallas import tpu as pltpu
from jax.experimental.pallas import tpu_sc as plsc

_NC = 2
_NS = 16
_NW = _NC * _NS
_NA = 2
_NB = 3


@functools.lru_cache(maxsize=None)
def _sc_edge_diff(E, N, D):
    epw = E // _NW
    half = epw // 2
    C = 40
    n_chunks = half // C
    stripe = (N // _NS) // 8 * 8
    last_stripe = N - (_NS - 1) * stripe

    mesh = plsc.VectorSubcoreMesh(core_axis_name="c", subcore_axis_name="s")

    @functools.partial(
        pl.kernel,
        mesh=mesh,
        out_type=jax.ShapeDtypeStruct((E, D), jnp.float32),
        scratch_types=[
            pltpu.VMEM((half,), jnp.int32),
            pltpu.VMEM((half,), jnp.int32),
            pltpu.VMEM((_NA, C, D), jnp.float32),
            pltpu.VMEM((_NA, C, D), jnp.float32),
            pltpu.VMEM((_NB * C,), jnp.int32),
            pltpu.VMEM((_NB * C,), jnp.int32),
            pltpu.VMEM((_NB, C, D), jnp.float32),
            pltpu.VMEM_SHARED((N, D), jnp.float32),
            pltpu.SemaphoreType.DMA,
            pltpu.SemaphoreType.DMA,
        ] + [pltpu.SemaphoreType.DMA] * (3 * _NA + 4 * _NB),
    )
    def k(x_hbm, xneg_hbm, src_hbm, dst_hbm, out_hbm,
          idx_as, idx_ad, rows_s, rows_d, idx_bs, idx_bd, bbuf, xneg_sp,
          sem_is, sem_id, *sems):
        sem_ag1 = sems[0:_NA]
        sem_ag2 = sems[_NA:2 * _NA]
        sem_ao = sems[2 * _NA:3 * _NA]
        rest = sems[3 * _NA:]
        sem_bi = rest[0:_NB]
        sem_bg1 = rest[_NB:2 * _NB]
        sem_bg2 = rest[2 * _NB:3 * _NB]
        sem_bo = rest[3 * _NB:4 * _NB]

        wid = lax.axis_index("s") * _NC + lax.axis_index("c")
        base_a = wid * epw
        base_b = base_a + half

        cp_is = pltpu.async_copy(
            src_hbm.at[pl.ds(base_a, half)], idx_as, sem_is)
        cp_id = pltpu.async_copy(
            dst_hbm.at[pl.ds(base_a, half)], idx_ad, sem_id)

        sid = lax.axis_index("s")

        @pl.when(sid < _NS - 1)
        def _():
            pltpu.sync_copy(xneg_hbm.at[pl.ds(sid * stripe, stripe)],
                            xneg_sp.at[pl.ds(sid * stripe, stripe)])

        @pl.when(sid == _NS - 1)
        def _():
            pltpu.sync_copy(xneg_hbm.at[pl.ds((_NS - 1) * stripe, last_stripe)],
                            xneg_sp.at[pl.ds((_NS - 1) * stripe, last_stripe)])

        plsc.subcore_barrier()

        cp_is.wait()
        cp_id.wait()

        def a_issue_gathers(c, b):
            pltpu.async_copy(
                xneg_sp.at[idx_as.at[pl.ds(c * C, C)]], rows_s.at[b],
                sem_ag1[b])
            pltpu.async_copy(
                xneg_sp.at[idx_ad.at[pl.ds(c * C, C)]], rows_d.at[b],
                sem_ag2[b])

        def a_wait_gathers(b):
            pltpu.make_async_copy(
                xneg_sp.at[idx_as.at[pl.ds(0, C)]], rows_s.at[b],
                sem_ag1[b]).wait()
            pltpu.make_async_copy(
                xneg_sp.at[idx_ad.at[pl.ds(0, C)]], rows_d.at[b],
                sem_ag2[b]).wait()

        def a_issue_write(c, b):
            pltpu.async_copy(
                rows_d.at[b], out_hbm.at[pl.ds(base_a + c * C, C)], sem_ao[b])

        def a_wait_write(b):
            pltpu.make_async_copy(
                rows_d.at[b], out_hbm.at[pl.ds(base_a, C)], sem_ao[b]).wait()

        def a_compute(b):
            RU = 4

            def row_body(r, rcarry):
                for rr in range(RU):
                    row = r * RU + rr
                    for v in range(D // 16):
                        sl = pl.ds(v * 16, 16)
                        rows_d[b, row, sl] = (
                            rows_s[b, row, sl] - rows_d[b, row, sl])
                return rcarry
            lax.fori_loop(0, C // RU, row_body, 0)

        def a_stage(c, b):
            a_wait_gathers(b)
            a_compute(b)
            a_issue_write(c, b)

            b1 = (b + 1) % _NA

            @pl.when(c + 1 < n_chunks)
            def _():
                @pl.when(c + 1 >= _NA)
                def _():
                    a_wait_write(b1)
                a_issue_gathers(c + 1, b1)

        def b_issue_idx(c, b):
            pltpu.async_copy(
                src_hbm.at[pl.ds(base_b + c * C, C)],
                idx_bs.at[pl.ds(b * C, C)], sem_bi[b])
            pltpu.async_copy(
                dst_hbm.at[pl.ds(base_b + c * C, C)],
                idx_bd.at[pl.ds(b * C, C)], sem_bi[b])

        def b_wait_idx(b):
            pltpu.make_async_copy(
                src_hbm.at[pl.ds(base_b, C)],
                idx_bs.at[pl.ds(b * C, C)], sem_bi[b]).wait()
            pltpu.make_async_copy(
                dst_hbm.at[pl.ds(base_b, C)],
                idx_bd.at[pl.ds(b * C, C)], sem_bi[b]).wait()

        def b_issue_g1(b):
            pltpu.async_copy(
                xneg_sp.at[idx_bs.at[pl.ds(b * C, C)]], bbuf.at[b],
                sem_bg1[b])

        def b_wait_g1(b):
            pltpu.make_async_copy(
                xneg_sp.at[idx_bs.at[pl.ds(0, C)]], bbuf.at[b],
                sem_bg1[b]).wait()

        def b_issue_g2(b):
            pltpu.async_copy(
                x_hbm.at[idx_bd.at[pl.ds(b * C, C)]], bbuf.at[b],
                sem_bg2[b], add=True)

        def b_wait_g2(b):
            pltpu.make_async_copy(
                x_hbm.at[idx_bd.at[pl.ds(0, C)]], bbuf.at[b],
                sem_bg2[b]).wait()

        def b_issue_write(c, b):
            pltpu.async_copy(
                bbuf.at[b], out_hbm.at[pl.ds(base_b + c * C, C)], sem_bo[b])

        def b_wait_write(b):
            pltpu.make_async_copy(
                bbuf.at[b], out_hbm.at[pl.ds(base_b, C)], sem_bo[b]).wait()

        def b_stage(c, b):
            b_wait_g2(b)
            b_issue_write(c, b)

            @pl.when(c + 3 < n_chunks)
            def _():
                b_issue_idx(c + 3, b)

            b2 = (b + 2) % _NB

            @pl.when(c + 2 < n_chunks)
            def _():
                b_wait_idx(b2)

                @pl.when(c + 2 >= _NB)
                def _():
                    b_wait_write(b2)
                b_issue_g1(b2)

            b1 = (b + 1) % _NB

            @pl.when(c + 1 < n_chunks)
            def _():
                b_wait_g1(b1)
                b_issue_g2(b1)

        a_issue_gathers(0, 0)

        b_issue_idx(0, 0)
        b_issue_idx(1, 1)
        b_issue_idx(2, 2)
        b_wait_idx(0)
        b_issue_g1(0)
        b_wait_idx(1)
        b_issue_g1(1)
        b_wait_g1(0)
        b_issue_g2(0)

        LCM = 6
        n_pipe = n_chunks - (n_chunks % LCM)

        def body(i, carry):
            for j in range(LCM):
                c = i * LCM + j
                a_stage(c, j % _NA)
                b_stage(c, j % _NB)
            return carry

        lax.fori_loop(0, n_pipe // LCM, body, 0)
        for c in range(n_pipe, n_chunks):
            a_stage(c, c % _NA)
            b_stage(c, c % _NB)

        for b in range(_NA):
            a_wait_write(b)
        for b in range(_NB):
            b_wait_write(b)

    return k


def kernel(x, edge_index):
    N, D = x.shape
    E = edge_index.shape[1]
    src = edge_index[0]
    dst = edge_index[1]
    return _sc_edge_diff(E, N, D)(x, -x, src, dst)
